# baseline (speedup 1.0000x reference)
import functools

import jax
import jax.numpy as jnp
from jax import lax
from jax.experimental import pallas as pl
from jax.experimental.pallas import tpu as pltpu
from jax.experimental.pallas import tpu_sc as plsc

_IDX_CHUNK = 128


def _make_sc_gather(vocab, n_turns, n_tok):
    info = plsc.get_sparse_core_info()
    nw = info.num_cores * info.num_subcores
    tok_per_w = n_tok // nw
    n_chunks = tok_per_w // _IDX_CHUNK
    mesh = plsc.VectorSubcoreMesh(core_axis_name="c", subcore_axis_name="s")

    @functools.partial(
        pl.kernel,
        mesh=mesh,
        compiler_params=pltpu.CompilerParams(use_tc_tiling_on_sc=False),
        out_type=jax.ShapeDtypeStruct((n_tok, n_turns), jnp.float32),
        scratch_types=[
            pltpu.VMEM((n_chunks, _IDX_CHUNK), jnp.int32),
            pltpu.VMEM((tok_per_w, n_turns), jnp.float32),
            pltpu.SemaphoreType.DMA,
        ],
    )
    def gather_kernel(turns_hbm, ids_hbm, out_hbm, idx_v, rows_v, sem):
        wid = lax.axis_index("s") * info.num_cores + lax.axis_index("c")
        base = wid * tok_per_w
        pltpu.sync_copy(ids_hbm.at[pl.ds(wid * n_chunks, n_chunks)], idx_v)
        copies = [
            pltpu.async_copy(
                turns_hbm.at[idx_v.at[c]],
                rows_v.at[pl.ds(c * _IDX_CHUNK, _IDX_CHUNK)],
                sem,
            )
            for c in range(n_chunks)
        ]
        for cp in copies:
            cp.wait()
        pltpu.sync_copy(rows_v, out_hbm.at[pl.ds(base, tok_per_w)])

    return gather_kernel


def _tc_body(x_ref, pc_ref, pos_ref, g_ref, b_ref, o_ref):
    x = x_ref[...]
    pc = pc_ref[...]
    t = x.shape[-1]
    x2 = x * x
    x3 = x2 * x
    x4 = x2 * x2
    c0 = jnp.sum(pc[0:t], axis=0, keepdims=True)
    emb = jnp.dot(x, pc[t : 2 * t], preferred_element_type=jnp.float32)
    emb = emb + jnp.dot(x2, pc[2 * t : 3 * t], preferred_element_type=jnp.float32)
    emb = emb + jnp.dot(x3, pc[3 * t : 4 * t], preferred_element_type=jnp.float32)
    emb = emb + jnp.dot(x4, pc[4 * t : 5 * t], preferred_element_type=jnp.float32)
    emb = emb + c0 + pos_ref[...]
    mean = jnp.mean(emb, axis=-1, keepdims=True)
    cen = emb - mean
    var = jnp.mean(cen * cen, axis=-1, keepdims=True)
    o_ref[...] = cen * lax.rsqrt(var + 1e-12) * g_ref[...] + b_ref[...]


_BS = 256


def _tc_dense(x, pc, pos_table, gamma, beta):
    b, s, t = x.shape
    d = pos_table.shape[-1]
    nsb = s // _BS
    x2d = x.reshape(b * s, t)
    return pl.pallas_call(
        _tc_body,
        grid=(nsb, b),
        in_specs=[
            pl.BlockSpec((_BS, t), lambda j, i: (i * nsb + j, 0)),
            pl.BlockSpec((pc.shape[0], d), lambda j, i: (0, 0)),
            pl.BlockSpec((_BS, d), lambda j, i: (j, 0)),
            pl.BlockSpec((1, d), lambda j, i: (0, 0)),
            pl.BlockSpec((1, d), lambda j, i: (0, 0)),
        ],
        out_specs=pl.BlockSpec((_BS, d), lambda j, i: (i * nsb + j, 0)),
        out_shape=jax.ShapeDtypeStruct((b * s, d), jnp.float32),
    )(x2d, pc, pos_table, gamma, beta).reshape(b, s, d)


def kernel(input_ids, turns, poly_coeffs, pos_table, ln_gamma, ln_beta):
    b, s = input_ids.shape
    vocab, n_turns = turns.shape
    n_tok = b * s
    ids = input_ids.astype(jnp.int32).reshape(n_tok // _IDX_CHUNK, _IDX_CHUNK)
    x = _make_sc_gather(vocab, n_turns, n_tok)(turns, ids)
    pc = jnp.transpose(poly_coeffs, (1, 0, 2)).reshape(-1, poly_coeffs.shape[-1])
    out = _tc_dense(
        x.reshape(b, s, n_turns),
        pc,
        pos_table,
        ln_gamma.reshape(1, -1),
        ln_beta.reshape(1, -1),
    )
    return out


# speedup vs baseline: 1.1265x; 1.1265x over previous
import functools

import jax
import jax.numpy as jnp
from jax import lax
from jax.experimental import pallas as pl
from jax.experimental.pallas import tpu as pltpu
from jax.experimental.pallas import tpu_sc as plsc

_IDX_CHUNK = 128


def _make_sc_gather(vocab, n_turns, n_tok):
    info = plsc.get_sparse_core_info()
    nw = info.num_cores * info.num_subcores
    tok_per_w = n_tok // nw
    n_chunks = tok_per_w // _IDX_CHUNK
    mesh = plsc.VectorSubcoreMesh(core_axis_name="c", subcore_axis_name="s")

    @functools.partial(
        pl.kernel,
        mesh=mesh,
        compiler_params=pltpu.CompilerParams(use_tc_tiling_on_sc=False),
        out_type=jax.ShapeDtypeStruct((n_tok, n_turns), jnp.float32),
        scratch_types=[
            pltpu.VMEM((n_chunks, _IDX_CHUNK), jnp.int32),
            pltpu.VMEM((tok_per_w, n_turns), jnp.float32),
            pltpu.SemaphoreType.DMA,
        ],
    )
    def gather_kernel(turns_hbm, ids_hbm, out_hbm, idx_v, rows_v, sem):
        wid = lax.axis_index("s") * info.num_cores + lax.axis_index("c")
        base = wid * tok_per_w
        pltpu.sync_copy(ids_hbm.at[pl.ds(wid * n_chunks, n_chunks)], idx_v)
        copies = [
            pltpu.async_copy(
                turns_hbm.at[idx_v.at[c]],
                rows_v.at[pl.ds(c * _IDX_CHUNK, _IDX_CHUNK)],
                sem,
            )
            for c in range(n_chunks)
        ]
        for cp in copies:
            cp.wait()
        pltpu.sync_copy(rows_v, out_hbm.at[pl.ds(base, tok_per_w)])

    return gather_kernel


def _tc_body(x_ref, pc_ref, pos_ref, g_ref, b_ref, o_ref):
    x = x_ref[...]
    pc = pc_ref[...]
    t = x.shape[-1]
    x2 = x * x
    x3 = x2 * x
    x4 = x2 * x2
    c0 = jnp.sum(pc[0:t], axis=0, keepdims=True)
    emb = jnp.dot(x, pc[t : 2 * t], preferred_element_type=jnp.float32)
    emb = emb + jnp.dot(x2, pc[2 * t : 3 * t], preferred_element_type=jnp.float32)
    emb = emb + jnp.dot(x3, pc[3 * t : 4 * t], preferred_element_type=jnp.float32)
    emb = emb + jnp.dot(x4, pc[4 * t : 5 * t], preferred_element_type=jnp.float32)
    emb = emb + c0 + pos_ref[...]
    mean = jnp.mean(emb, axis=-1, keepdims=True)
    cen = emb - mean
    var = jnp.mean(cen * cen, axis=-1, keepdims=True)
    o_ref[...] = cen * lax.rsqrt(var + 1e-12) * g_ref[...] + b_ref[...]


_BS = 1024


def _tc_dense(x, pc, pos_table, gamma, beta):
    b, s, t = x.shape
    d = pos_table.shape[-1]
    nsb = s // _BS
    x2d = x.reshape(b * s, t)
    return pl.pallas_call(
        _tc_body,
        grid=(nsb, b),
        in_specs=[
            pl.BlockSpec((_BS, t), lambda j, i: (i * nsb + j, 0)),
            pl.BlockSpec((pc.shape[0], d), lambda j, i: (0, 0)),
            pl.BlockSpec((_BS, d), lambda j, i: (j, 0)),
            pl.BlockSpec((1, d), lambda j, i: (0, 0)),
            pl.BlockSpec((1, d), lambda j, i: (0, 0)),
        ],
        out_specs=pl.BlockSpec((_BS, d), lambda j, i: (i * nsb + j, 0)),
        out_shape=jax.ShapeDtypeStruct((b * s, d), jnp.float32),
        compiler_params=pltpu.CompilerParams(
            dimension_semantics=("parallel", "parallel"),
        ),
    )(x2d, pc, pos_table, gamma, beta).reshape(b, s, d)


def kernel(input_ids, turns, poly_coeffs, pos_table, ln_gamma, ln_beta):
    b, s = input_ids.shape
    vocab, n_turns = turns.shape
    n_tok = b * s
    ids = input_ids.astype(jnp.int32).reshape(n_tok // _IDX_CHUNK, _IDX_CHUNK)
    x = _make_sc_gather(vocab, n_turns, n_tok)(turns, ids)
    pc = jnp.transpose(poly_coeffs, (1, 0, 2)).reshape(-1, poly_coeffs.shape[-1])
    out = _tc_dense(
        x.reshape(b, s, n_turns),
        pc,
        pos_table,
        ln_gamma.reshape(1, -1),
        ln_beta.reshape(1, -1),
    )
    return out


# speedup vs baseline: 1.2620x; 1.1202x over previous
import functools

import jax
import jax.numpy as jnp
from jax import lax
from jax.experimental import pallas as pl
from jax.experimental.pallas import tpu as pltpu
from jax.experimental.pallas import tpu_sc as plsc

_IDX_CHUNK = 128
_NPOW = 4


def _make_sc_gather_powers(n_turns, n_tok):
    info = plsc.get_sparse_core_info()
    nw = info.num_cores * info.num_subcores
    npw = _NPOW * n_turns
    tok_per_w = n_tok // nw
    n_chunks = tok_per_w // _IDX_CHUNK
    lanes = info.num_lanes
    pairs = tok_per_w // (lanes // n_turns)
    mesh = plsc.VectorSubcoreMesh(core_axis_name="c", subcore_axis_name="s")

    @functools.partial(
        pl.kernel,
        mesh=mesh,
        compiler_params=pltpu.CompilerParams(
            use_tc_tiling_on_sc=False, needs_layout_passes=False
        ),
        out_type=jax.ShapeDtypeStruct((n_tok, npw), jnp.float32),
        scratch_types=[
            pltpu.VMEM((n_chunks, _IDX_CHUNK), jnp.int32),
            pltpu.VMEM((tok_per_w, n_turns), jnp.float32),
            pltpu.VMEM((tok_per_w, npw), jnp.float32),
            pltpu.SemaphoreType.DMA,
        ],
    )
    def gather_kernel(turns_hbm, ids_hbm, out_hbm, idx_v, rows_v, pow_v, sem):
        wid = lax.axis_index("s") * info.num_cores + lax.axis_index("c")
        base = wid * tok_per_w
        pltpu.sync_copy(ids_hbm.at[pl.ds(wid * n_chunks, n_chunks)], idx_v)
        copies = [
            pltpu.async_copy(
                turns_hbm.at[idx_v.at[c]],
                rows_v.at[pl.ds(c * _IDX_CHUNK, _IDX_CHUNK)],
                sem,
            )
            for c in range(n_chunks)
        ]
        for cp in copies:
            cp.wait()

        lane = lax.broadcasted_iota(jnp.int32, (lanes,), 0)
        half = lane >> 3
        within = lane & (n_turns - 1)
        cols = [within + d * n_turns for d in range(_NPOW)]

        def body(k, _):
            row = 2 * k + half
            x = plsc.load_gather(rows_v, [row, within])
            x2 = x * x
            x3 = x2 * x
            x4 = x2 * x2
            for d, v in enumerate((x, x2, x3, x4)):
                plsc.store_scatter(pow_v, [row, cols[d]], v)
            return 0

        lax.fori_loop(0, pairs, body, 0)
        pltpu.sync_copy(pow_v, out_hbm.at[pl.ds(base, tok_per_w)])

    return gather_kernel


def _tc_body(p_ref, pc_ref, pos_ref, g_ref, b_ref, o_ref):
    p = p_ref[...]
    pc = pc_ref[...]
    t = pc.shape[0] - p.shape[-1]
    c0 = jnp.sum(pc[0:t], axis=0, keepdims=True)
    emb = jnp.dot(p, pc[t:], preferred_element_type=jnp.float32)
    emb = emb + c0 + pos_ref[...]
    mean = jnp.mean(emb, axis=-1, keepdims=True)
    cen = emb - mean
    var = jnp.mean(cen * cen, axis=-1, keepdims=True)
    o_ref[...] = cen * lax.rsqrt(var + 1e-12) * g_ref[...] + b_ref[...]


_BS = 1024


def _tc_dense(p2d, pc, pos_table, gamma, beta, b, s):
    d = pos_table.shape[-1]
    nsb = s // _BS
    npw = p2d.shape[-1]
    return pl.pallas_call(
        _tc_body,
        grid=(nsb, b),
        in_specs=[
            pl.BlockSpec((_BS, npw), lambda j, i: (i * nsb + j, 0)),
            pl.BlockSpec((pc.shape[0], d), lambda j, i: (0, 0)),
            pl.BlockSpec((_BS, d), lambda j, i: (j, 0)),
            pl.BlockSpec((1, d), lambda j, i: (0, 0)),
            pl.BlockSpec((1, d), lambda j, i: (0, 0)),
        ],
        out_specs=pl.BlockSpec((_BS, d), lambda j, i: (i * nsb + j, 0)),
        out_shape=jax.ShapeDtypeStruct((b * s, d), jnp.float32),
        compiler_params=pltpu.CompilerParams(
            dimension_semantics=("parallel", "parallel"),
        ),
    )(p2d, pc, pos_table, gamma, beta).reshape(b, s, d)


def kernel(input_ids, turns, poly_coeffs, pos_table, ln_gamma, ln_beta):
    b, s = input_ids.shape
    vocab, n_turns = turns.shape
    n_tok = b * s
    ids = input_ids.astype(jnp.int32).reshape(n_tok // _IDX_CHUNK, _IDX_CHUNK)
    p2d = _make_sc_gather_powers(n_turns, n_tok)(turns, ids)
    pc = jnp.transpose(poly_coeffs, (1, 0, 2)).reshape(-1, poly_coeffs.shape[-1])
    return _tc_dense(
        p2d,
        pc,
        pos_table,
        ln_gamma.reshape(1, -1),
        ln_beta.reshape(1, -1),
        b,
        s,
    )


# speedup vs baseline: 2.1563x; 1.7087x over previous
import functools

import jax
import jax.numpy as jnp
from jax import lax
from jax.experimental import pallas as pl
from jax.experimental.pallas import tpu as pltpu
from jax.experimental.pallas import tpu_sc as plsc

_IDX_CHUNK = 128
_NPOW = 4


def _make_sc_gather_powers(n_turns, n_tok, vocab):
    info = plsc.get_sparse_core_info()
    nw = info.num_cores * info.num_subcores
    npw = _NPOW * n_turns
    tok_per_w = n_tok // nw
    lanes = info.num_lanes
    wpw = tok_per_w * n_turns
    n_chunks = wpw // _IDX_CHUNK
    pairs = tok_per_w // (lanes // n_turns)
    mesh = plsc.VectorSubcoreMesh(core_axis_name="c", subcore_axis_name="s")

    @functools.partial(
        pl.kernel,
        mesh=mesh,
        compiler_params=pltpu.CompilerParams(
            use_tc_tiling_on_sc=False, needs_layout_passes=False
        ),
        out_type=jax.ShapeDtypeStruct((n_tok, npw), jnp.float32),
        scratch_types=[
            pltpu.VMEM((tok_per_w,), jnp.int32),
            pltpu.VMEM((wpw,), jnp.int32),
            pltpu.VMEM((wpw,), jnp.float32),
            pltpu.VMEM((tok_per_w, npw), jnp.float32),
            pltpu.SemaphoreType.DMA,
        ],
    )
    def gather_kernel(turns_hbm, ids_hbm, out_hbm, ids_v, widx_v, rows_v, pow_v, sem):
        wid = lax.axis_index("s") * info.num_cores + lax.axis_index("c")
        base = wid * tok_per_w
        pltpu.sync_copy(ids_hbm.at[pl.ds(base, tok_per_w)], ids_v)

        lane = lax.broadcasted_iota(jnp.int32, (lanes,), 0)
        half = lane >> 3
        within = lane & (n_turns - 1)
        word_off = within * vocab
        cols = [within + d * n_turns for d in range(_NPOW)]

        def idx_body(k, _):
            row = 2 * k + half
            tok_id = plsc.load_gather(ids_v, [row])
            widx_v[pl.ds(k * lanes, lanes)] = tok_id + word_off
            return 0

        lax.fori_loop(0, pairs, idx_body, 0)

        copies = [
            pltpu.async_copy(
                turns_hbm.at[widx_v.at[pl.ds(c * _IDX_CHUNK, _IDX_CHUNK)]],
                rows_v.at[pl.ds(c * _IDX_CHUNK, _IDX_CHUNK)],
                sem,
            )
            for c in range(n_chunks)
        ]
        for cp in copies:
            cp.wait()

        def pow_body(k, _):
            row = 2 * k + half
            x = rows_v[pl.ds(k * lanes, lanes)]
            x2 = x * x
            x3 = x2 * x
            x4 = x2 * x2
            for d, v in enumerate((x, x2, x3, x4)):
                plsc.store_scatter(pow_v, [row, cols[d]], v)
            return 0

        lax.fori_loop(0, pairs, pow_body, 0)
        pltpu.sync_copy(pow_v, out_hbm.at[pl.ds(base, tok_per_w)])

    return gather_kernel


def _tc_body(p_ref, pc_ref, pos_ref, g_ref, b_ref, o_ref):
    p = p_ref[...]
    pc = pc_ref[...]
    t = pc.shape[0] - p.shape[-1]
    c0 = jnp.sum(pc[0:t], axis=0, keepdims=True)
    emb = jnp.dot(p, pc[t:], preferred_element_type=jnp.float32)
    emb = emb + c0 + pos_ref[...]
    mean = jnp.mean(emb, axis=-1, keepdims=True)
    cen = emb - mean
    var = jnp.mean(cen * cen, axis=-1, keepdims=True)
    o_ref[...] = cen * lax.rsqrt(var + 1e-12) * g_ref[...] + b_ref[...]


_BS = 1024


def _tc_dense(p2d, pc, pos_table, gamma, beta, b, s):
    d = pos_table.shape[-1]
    nsb = s // _BS
    npw = p2d.shape[-1]
    return pl.pallas_call(
        _tc_body,
        grid=(nsb, b),
        in_specs=[
            pl.BlockSpec((_BS, npw), lambda j, i: (i * nsb + j, 0)),
            pl.BlockSpec((pc.shape[0], d), lambda j, i: (0, 0)),
            pl.BlockSpec((_BS, d), lambda j, i: (j, 0)),
            pl.BlockSpec((1, d), lambda j, i: (0, 0)),
            pl.BlockSpec((1, d), lambda j, i: (0, 0)),
        ],
        out_specs=pl.BlockSpec((_BS, d), lambda j, i: (i * nsb + j, 0)),
        out_shape=jax.ShapeDtypeStruct((b * s, d), jnp.float32),
        compiler_params=pltpu.CompilerParams(
            dimension_semantics=("parallel", "parallel"),
        ),
    )(p2d, pc, pos_table, gamma, beta).reshape(b, s, d)


def kernel(input_ids, turns, poly_coeffs, pos_table, ln_gamma, ln_beta):
    b, s = input_ids.shape
    vocab, n_turns = turns.shape
    n_tok = b * s
    ids = input_ids.astype(jnp.int32).reshape(n_tok)
    turns_flat = turns.T.reshape(-1)
    p2d = _make_sc_gather_powers(n_turns, n_tok, vocab)(turns_flat, ids)
    pc = jnp.transpose(poly_coeffs, (1, 0, 2)).reshape(-1, poly_coeffs.shape[-1])
    return _tc_dense(
        p2d,
        pc,
        pos_table,
        ln_gamma.reshape(1, -1),
        ln_beta.reshape(1, -1),
        b,
        s,
    )
